# baseline (speedup 1.0000x reference)
import functools

import numpy as np
import jax
import jax.numpy as jnp
from jax import lax
from jax.experimental import pallas as pl
from jax.experimental.pallas import tpu as pltpu
from jax.experimental.pallas import tpu_sc as plsc

LAT = 16
WROWS = 100
OFF_A, OFF_B, OFF_BP, OFF_C, OFF_CP, OFF_D = 0, 16, 32, 48, 64, 80
OFF_BD0, OFF_BD, OFF_BU, OFF_BU0 = 96, 97, 98, 99

def _bcast(v, j):
    idx = lax.broadcast_in_dim(jnp.int32(j), (LAT,), ())
    return v.at[idx].get(mode="promise_in_bounds")


def _matvec16(w, off, v, accs):
    for j in range(LAT):
        t = w[off + j] * _bcast(v, j)
        accs[j % 4] = t if accs[j % 4] is None else accs[j % 4] + t


def _acc_sum(accs):
    return (accs[0] + accs[1]) + (accs[2] + accs[3])


def _chain_step(w, a_off, b_off, bias_off, v_self, v_par):
    accs = [w[bias_off], None, None, None]
    _matvec16(w, b_off, v_par, accs)
    _matvec16(w, a_off, v_self, accs)
    return 1.0 / (1.0 + jnp.exp(_acc_sum(accs)))


def _conv_body(w):
    def body(_, xs):
        x0, x1, x2, x3, x4, x5, x6, x7 = xs
        x1 = _chain_step(w, OFF_A, OFF_BP, OFF_BD0, x1, x0)
        x2 = _chain_step(w, OFF_A, OFF_B, OFF_BD, x2, x1)
        x3 = _chain_step(w, OFF_A, OFF_B, OFF_BD, x3, x2)
        x4 = _chain_step(w, OFF_A, OFF_B, OFF_BD, x4, x3)
        x5 = _chain_step(w, OFF_A, OFF_B, OFF_BD, x5, x4)
        x6 = _chain_step(w, OFF_A, OFF_B, OFF_BD, x6, x5)
        x7 = _chain_step(w, OFF_A, OFF_B, OFF_BD, x7, x6)
        x6 = _chain_step(w, OFF_C, OFF_D, OFF_BU, x6, x7)
        x5 = _chain_step(w, OFF_C, OFF_D, OFF_BU, x5, x6)
        x4 = _chain_step(w, OFF_C, OFF_D, OFF_BU, x4, x5)
        x3 = _chain_step(w, OFF_C, OFF_D, OFF_BU, x3, x4)
        x2 = _chain_step(w, OFF_C, OFF_D, OFF_BU, x2, x3)
        x1 = _chain_step(w, OFF_C, OFF_D, OFF_BU, x1, x2)
        x0 = _chain_step(w, OFF_CP, OFF_D, OFF_BU0, x0, x1)
        return (x0, x1, x2, x3, x4, x5, x6, x7)

    return body


@functools.lru_cache(maxsize=None)
def _make_run(n_pass):
    mesh = plsc.VectorSubcoreMesh(core_axis_name="c", subcore_axis_name="s")

    @functools.partial(
        pl.kernel,
        mesh=mesh,
        out_type=jax.ShapeDtypeStruct((LAT,), jnp.float32),
        scratch_types=[
            pltpu.VMEM((8, LAT), jnp.float32),
            pltpu.VMEM((WROWS, LAT), jnp.float32),
            pltpu.VMEM((WROWS, LAT), jnp.float32),
        ],
    )
    def run(x8_hbm, ws_hbm, wc_hbm, out_hbm, x_v, ws_v, wc_v):
        first = jnp.logical_and(lax.axis_index("c") == 0, lax.axis_index("s") == 0)

        @pl.when(first)
        def _():
            pltpu.sync_copy(x8_hbm, x_v)
            pltpu.sync_copy(ws_hbm, ws_v)
            pltpu.sync_copy(wc_hbm, wc_v)
            xs = tuple(x_v[i] for i in range(8))
            xs = lax.fori_loop(0, n_pass, _conv_body(ws_v), xs)
            xs = lax.fori_loop(0, n_pass, _conv_body(wc_v), xs)
            x_v[0, :] = xs[0]
            pltpu.sync_copy(x_v.at[0], out_hbm)

    return run


def _pack(Wr, br, Wd, bd, Wu, bu):
    Wd1, Wd2 = Wd[:, :LAT], Wd[:, LAT:]
    Wu1, Wu2 = Wu[:, :LAT], Wu[:, LAT:]
    return -jnp.concatenate(
        [
            (Wd1 @ Wr).T,
            Wd2.T,
            (Wd2 @ Wr).T,
            Wu1.T,
            (Wu1 @ Wr).T,
            Wu2.T,
            (bd + Wd1 @ br + Wd2 @ br)[None],
            (bd + Wd1 @ br)[None],
            bu[None],
            (bu + Wu1 @ br)[None],
        ],
        axis=0,
    )


def kernel(x, num_node, edge_index, Ws_root, bs_root, Ws_down, bs_down, Ws_up, bs_up,
           Wc_root, bc_root, Wc_down, bc_down, Wc_up, bc_up):
    N = x.shape[0]
    x8 = jnp.pad(x[:8], ((0, 0), (0, LAT - x.shape[1])))
    ws = _pack(Ws_root, bs_root, Ws_down, bs_down, Ws_up, bs_up)
    wc = _pack(Wc_root, bc_root, Wc_down, bc_down, Wc_up, bc_up)
    return _make_run(N - 1)(x8, ws, wc)


# speedup vs baseline: 1.0957x; 1.0957x over previous
import functools

import numpy as np
import jax
import jax.numpy as jnp
from jax import lax
from jax.experimental import pallas as pl
from jax.experimental.pallas import tpu as pltpu
from jax.experimental.pallas import tpu_sc as plsc

LAT = 16
WROWS = 100
OFF_A, OFF_B, OFF_BP, OFF_C, OFF_CP, OFF_D = 0, 16, 32, 48, 64, 80
OFF_BD0, OFF_BD, OFF_BU, OFF_BU0 = 96, 97, 98, 99

def _bcast(v, j):
    idx = lax.broadcast_in_dim(jnp.int32(j), (LAT,), ())
    return v.at[idx].get(mode="promise_in_bounds")


def _matvec16(w, off, v, accs):
    for j in range(LAT):
        t = w[off + j] * _bcast(v, j)
        accs[j % 4] = t if accs[j % 4] is None else accs[j % 4] + t


def _acc_sum(accs):
    return (accs[0] + accs[1]) + (accs[2] + accs[3])


def _chain_step(w, a_off, b_off, bias_off, v_self, v_par):
    accs = [w[bias_off], None, None, None]
    _matvec16(w, a_off, v_self, accs)
    _matvec16(w, b_off, v_par, accs)
    return 1.0 / (1.0 + jnp.exp(_acc_sum(accs)))


def _conv_body(w):
    def body(_, xs):
        x0, x1, x2, x3, x4, x5, x6, x7 = xs
        x1 = _chain_step(w, OFF_A, OFF_BP, OFF_BD0, x1, x0)
        x2 = _chain_step(w, OFF_A, OFF_B, OFF_BD, x2, x1)
        x3 = _chain_step(w, OFF_A, OFF_B, OFF_BD, x3, x2)
        x4 = _chain_step(w, OFF_A, OFF_B, OFF_BD, x4, x3)
        x5 = _chain_step(w, OFF_A, OFF_B, OFF_BD, x5, x4)
        x6 = _chain_step(w, OFF_A, OFF_B, OFF_BD, x6, x5)
        x7 = _chain_step(w, OFF_A, OFF_B, OFF_BD, x7, x6)
        x6 = _chain_step(w, OFF_C, OFF_D, OFF_BU, x6, x7)
        x5 = _chain_step(w, OFF_C, OFF_D, OFF_BU, x5, x6)
        x4 = _chain_step(w, OFF_C, OFF_D, OFF_BU, x4, x5)
        x3 = _chain_step(w, OFF_C, OFF_D, OFF_BU, x3, x4)
        x2 = _chain_step(w, OFF_C, OFF_D, OFF_BU, x2, x3)
        x1 = _chain_step(w, OFF_C, OFF_D, OFF_BU, x1, x2)
        x0 = _chain_step(w, OFF_CP, OFF_D, OFF_BU0, x0, x1)
        return (x0, x1, x2, x3, x4, x5, x6, x7)

    return body


@functools.lru_cache(maxsize=None)
def _make_run(n_pass):
    mesh = plsc.VectorSubcoreMesh(core_axis_name="c", subcore_axis_name="s")

    @functools.partial(
        pl.kernel,
        mesh=mesh,
        out_type=jax.ShapeDtypeStruct((LAT,), jnp.float32),
        scratch_types=[
            pltpu.VMEM((8, LAT), jnp.float32),
            pltpu.VMEM((WROWS, LAT), jnp.float32),
            pltpu.VMEM((WROWS, LAT), jnp.float32),
        ],
    )
    def run(x8_hbm, ws_hbm, wc_hbm, out_hbm, x_v, ws_v, wc_v):
        first = jnp.logical_and(lax.axis_index("c") == 0, lax.axis_index("s") == 0)

        @pl.when(first)
        def _():
            pltpu.sync_copy(x8_hbm, x_v)
            pltpu.sync_copy(ws_hbm, ws_v)
            pltpu.sync_copy(wc_hbm, wc_v)
            xs = tuple(x_v[i] for i in range(8))
            xs = lax.fori_loop(0, n_pass, _conv_body(ws_v), xs)
            xs = lax.fori_loop(0, n_pass, _conv_body(wc_v), xs)
            x_v[0, :] = xs[0]
            pltpu.sync_copy(x_v.at[0], out_hbm)

    return run


def _pack(Wr, br, Wd, bd, Wu, bu):
    Wd1, Wd2 = Wd[:, :LAT], Wd[:, LAT:]
    Wu1, Wu2 = Wu[:, :LAT], Wu[:, LAT:]
    return -jnp.concatenate(
        [
            (Wd1 @ Wr).T,
            Wd2.T,
            (Wd2 @ Wr).T,
            Wu1.T,
            (Wu1 @ Wr).T,
            Wu2.T,
            (bd + Wd1 @ br + Wd2 @ br)[None],
            (bd + Wd1 @ br)[None],
            bu[None],
            (bu + Wu1 @ br)[None],
        ],
        axis=0,
    )


def kernel(x, num_node, edge_index, Ws_root, bs_root, Ws_down, bs_down, Ws_up, bs_up,
           Wc_root, bc_root, Wc_down, bc_down, Wc_up, bc_up):
    N = x.shape[0]
    x8 = jnp.pad(x[:8], ((0, 0), (0, LAT - x.shape[1])))
    ws = _pack(Ws_root, bs_root, Ws_down, bs_down, Ws_up, bs_up)
    wc = _pack(Wc_root, bc_root, Wc_down, bc_down, Wc_up, bc_up)
    return _make_run(N - 1)(x8, ws, wc)
